# baseline (speedup 1.0000x reference)
import functools

import jax
import jax.numpy as jnp
from jax import lax
from jax.experimental import pallas as pl
from jax.experimental.pallas import tpu as pltpu
from jax.experimental.pallas import tpu_sc as plsc

_B = 1024
_T_IN = 1024
_N_ST = 2048
_N_OUT = 1024
_HASH = 8192
_NBITS = 13
_NW = 32
_RING = 4
_TILE_N = 16


def _addr_body(x_ref, conn_ref, out_ref, *, block_n, threshold):
    x = x_ref[...]
    if threshold:
        xb = (x > 0.5).astype(jnp.bfloat16)
    else:
        xb = x.astype(jnp.bfloat16)
    conn = conn_ref[...]
    t = x.shape[1]
    t_iota = lax.broadcasted_iota(jnp.int32, (block_n, t), 1)
    wlo = jnp.zeros((block_n, t), jnp.int32)
    whi = jnp.zeros((block_n, t), jnp.int32)
    for k in range(7):
        wlo = wlo + jnp.where(t_iota == conn[:, k][:, None], 1 << k, 0)
    for k in range(7, _NBITS):
        whi = whi + jnp.where(t_iota == conn[:, k][:, None], 1 << (k - 7), 0)
    dn = (((1,), (1,)), ((), ()))
    lo = lax.dot_general(wlo.astype(jnp.bfloat16), xb, dn,
                         preferred_element_type=jnp.float32)
    hi = lax.dot_general(whi.astype(jnp.bfloat16), xb, dn,
                         preferred_element_type=jnp.float32)
    out_ref[...] = lo.astype(jnp.int32) + hi.astype(jnp.int32) * 128


def _make_addr_call(n_neurons, block_n, x_shape, x_dtype, threshold):
    return pl.pallas_call(
        functools.partial(_addr_body, block_n=block_n, threshold=threshold),
        grid=(n_neurons // block_n,),
        in_specs=[
            pl.BlockSpec(x_shape, lambda j: (0, 0)),
            pl.BlockSpec((block_n, 24), lambda j: (j, 0)),
        ],
        out_specs=pl.BlockSpec((block_n, _B), lambda j: (j, 0)),
        out_shape=jax.ShapeDtypeStruct((n_neurons, _B), jnp.int32),
    )


def _concat_body(win_ref, out_ref):
    out_ref[:, :_T_IN] = win_ref[...].astype(jnp.float32)
    out_ref[:, _T_IN:] = jnp.zeros((_B, _N_ST), jnp.float32)


_concat_call = pl.pallas_call(
    _concat_body,
    out_shape=jax.ShapeDtypeStruct((_B, _T_IN + _N_ST), jnp.float32),
)


@functools.lru_cache(maxsize=None)
def _make_sc_gather(n_neurons):
    per_w = n_neurons // _NW
    n_tiles = per_w // _TILE_N
    mesh = plsc.VectorSubcoreMesh(core_axis_name="c", subcore_axis_name="s")

    @functools.partial(
        pl.kernel,
        mesh=mesh,
        out_type=jax.ShapeDtypeStruct((_B, n_neurons), jnp.float32),
        compiler_params=pltpu.CompilerParams(needs_layout_passes=False,
                                             use_tc_tiling_on_sc=False),
        scratch_types=(
            [pltpu.VMEM((_HASH,), jnp.float32) for _ in range(_RING)]
            + [pltpu.VMEM((_B,), jnp.int32) for _ in range(_RING)]
            + [pltpu.VMEM((_B, _TILE_N), jnp.float32) for _ in range(2)]
            + [
                pltpu.SemaphoreType.DMA,
                pltpu.SemaphoreType.DMA,
                pltpu.SemaphoreType.DMA,
            ]
        ),
    )
    def gather_kernel(table_hbm, addr_hbm, out_hbm, *scratch):
        rows_v = scratch[:_RING]
        idx_v = scratch[_RING:2 * _RING]
        tile_v = scratch[2 * _RING:2 * _RING + 2]
        rsem, isem, osem = scratch[2 * _RING + 2:]
        c = lax.axis_index("c")
        s = lax.axis_index("s")
        base = (s * 2 + c) * per_w

        for p in range(_RING - 1):
            pltpu.async_copy(table_hbm.at[base + p], rows_v[p], rsem)
            pltpu.async_copy(addr_hbm.at[base + p], idx_v[p], isem)

        row_iota = lax.iota(jnp.int32, 16)

        for tile in range(n_tiles):
            tv = tile_v[tile % 2]
            if tile >= 2:
                pltpu.make_async_copy(
                    tv, out_hbm.at[:, pl.ds(0, _TILE_N)], osem).wait()

            def group(g, carry, *, tile=tile, tv=tv):
                for sl in range(_RING):
                    j = g * _RING + sl
                    i = tile * _TILE_N + j
                    nf = i + _RING - 1

                    @pl.when(nf < per_w)
                    def _():
                        pltpu.async_copy(table_hbm.at[base + nf],
                                         rows_v[(sl - 1) % _RING], rsem)
                        pltpu.async_copy(addr_hbm.at[base + nf],
                                         idx_v[(sl - 1) % _RING], isem)

                    pltpu.make_async_copy(table_hbm.at[base + i],
                                          rows_v[sl], rsem).wait()
                    pltpu.make_async_copy(addr_hbm.at[base + i],
                                          idx_v[sl], isem).wait()

                    row = rows_v[sl]
                    idx = idx_v[sl]
                    col = jnp.full((16,), j, jnp.int32)

                    def gather16(v, _):
                        for u in range(8):
                            off = v * 128 + u * 16
                            cols = idx[pl.ds(off, 16)]
                            vals = plsc.load_gather(row, [cols])
                            plsc.store_scatter(tv, [row_iota + off, col], vals)
                        return _

                    lax.fori_loop(0, _B // 128, gather16, 0, unroll=True)
                return carry

            lax.fori_loop(0, _TILE_N // _RING, group, 0)
            pltpu.async_copy(
                tv, out_hbm.at[:, pl.ds(base + tile * _TILE_N, _TILE_N)],
                osem)

        for _ in range(min(n_tiles, 2)):
            pltpu.make_async_copy(
                tile_v[0], out_hbm.at[:, pl.ds(0, _TILE_N)], osem).wait()

    return gather_kernel


_addr1_call = _make_addr_call(_N_ST, 256, (_B, _T_IN), jnp.int32, False)
_addr2_call = _make_addr_call(_N_OUT, 256, (_B, _N_ST), jnp.float32, True)


def kernel(window_bits, conn_state, conn_out, state_memory, output_memory):
    state_layer_input = _concat_call(window_bits)

    addr1t = _addr1_call(window_bits, conn_state)
    state_layer_output = _make_sc_gather(_N_ST)(state_memory, addr1t)

    addr2t = _addr2_call(state_layer_output, conn_out)
    output_layer_output = _make_sc_gather(_N_OUT)(output_memory, addr2t)

    return (state_layer_input, state_layer_output, state_layer_output,
            output_layer_output)


# speedup vs baseline: 1.7486x; 1.7486x over previous
import functools

import jax
import jax.numpy as jnp
from jax import lax
from jax.experimental import pallas as pl
from jax.experimental.pallas import tpu as pltpu
from jax.experimental.pallas import tpu_sc as plsc

_B = 1024
_T_IN = 1024
_N_ST = 2048
_N_OUT = 1024
_HASH = 8192
_NBITS = 13
_NW = 32
_CH = 4
_RING = 2


def _build_w(conn, block_n, t):
    t_iota = lax.broadcasted_iota(jnp.int32, (block_n, t), 1)
    wlo = jnp.zeros((block_n, t), jnp.int32)
    whi = jnp.zeros((block_n, t), jnp.int32)
    for k in range(7):
        wlo = wlo + jnp.where(t_iota == conn[:, k][:, None], 1 << k, 0)
    for k in range(7, _NBITS):
        whi = whi + jnp.where(t_iota == conn[:, k][:, None], 1 << (k - 7), 0)
    return wlo.astype(jnp.bfloat16), whi.astype(jnp.bfloat16)


def _addr1_body(x_ref, conn_ref, out_ref, *, block_n):
    xb = x_ref[...].astype(jnp.bfloat16)
    wlo, whi = _build_w(conn_ref[...], block_n, _T_IN)
    dn = (((1,), (1,)), ((), ()))
    lo = lax.dot_general(wlo, xb, dn, preferred_element_type=jnp.float32)
    hi = lax.dot_general(whi, xb, dn, preferred_element_type=jnp.float32)
    out_ref[...] = lo.astype(jnp.int32) + hi.astype(jnp.int32) * 128


_addr1_call = pl.pallas_call(
    functools.partial(_addr1_body, block_n=256),
    grid=(_N_ST // 256,),
    in_specs=[
        pl.BlockSpec((_B, _T_IN), lambda j: (0, 0)),
        pl.BlockSpec((256, 24), lambda j: (j, 0)),
    ],
    out_specs=pl.BlockSpec((256, _B), lambda j: (j, 0)),
    out_shape=jax.ShapeDtypeStruct((_N_ST, _B), jnp.int32),
)


def _w2_body(conn_ref, wlo_ref, whi_ref, *, block_n):
    wlo, whi = _build_w(conn_ref[...], block_n, _N_ST)
    wlo_ref[...] = wlo
    whi_ref[...] = whi


_w2_call = pl.pallas_call(
    functools.partial(_w2_body, block_n=256),
    grid=(_N_OUT // 256,),
    in_specs=[pl.BlockSpec((256, 24), lambda j: (j, 0))],
    out_specs=[pl.BlockSpec((256, _N_ST), lambda j: (j, 0))] * 2,
    out_shape=[jax.ShapeDtypeStruct((_N_OUT, _N_ST), jnp.bfloat16)] * 2,
)


def _addr2_body(x_ref, wlo_ref, whi_ref, out_ref):
    xb = (x_ref[...] > 0.5).astype(jnp.bfloat16)
    dn = (((1,), (0,)), ((), ()))
    lo = lax.dot_general(wlo_ref[...], xb, dn,
                         preferred_element_type=jnp.float32)
    hi = lax.dot_general(whi_ref[...], xb, dn,
                         preferred_element_type=jnp.float32)
    out_ref[...] = lo.astype(jnp.int32) + hi.astype(jnp.int32) * 128


_addr2_call = pl.pallas_call(
    _addr2_body,
    grid=(_N_OUT // 256,),
    in_specs=[
        pl.BlockSpec((_N_ST, _B), lambda j: (0, 0)),
        pl.BlockSpec((256, _N_ST), lambda j: (j, 0)),
        pl.BlockSpec((256, _N_ST), lambda j: (j, 0)),
    ],
    out_specs=pl.BlockSpec((256, _B), lambda j: (j, 0)),
    out_shape=jax.ShapeDtypeStruct((_N_OUT, _B), jnp.int32),
)


def _concat_body(win_ref, out_ref):
    out_ref[:, :_T_IN] = win_ref[...].astype(jnp.float32)
    out_ref[:, _T_IN:] = jnp.zeros((_B, _N_ST), jnp.float32)


_concat_call = pl.pallas_call(
    _concat_body,
    out_shape=jax.ShapeDtypeStruct((_B, _T_IN + _N_ST), jnp.float32),
)


def _transpose_body(x_ref, out_ref):
    out_ref[...] = x_ref[...].T


def _make_transpose(n_rows):
    blk = 512
    return pl.pallas_call(
        _transpose_body,
        grid=(n_rows // blk, _B // blk),
        in_specs=[pl.BlockSpec((blk, blk), lambda i, j: (i, j))],
        out_specs=pl.BlockSpec((blk, blk), lambda i, j: (j, i)),
        out_shape=jax.ShapeDtypeStruct((_B, n_rows), jnp.float32),
    )


@functools.lru_cache(maxsize=None)
def _make_sc_gather(n_neurons):
    per_w = n_neurons // _NW
    n_chunks = per_w // _CH
    assert n_chunks % _RING == 0
    mesh = plsc.VectorSubcoreMesh(core_axis_name="c", subcore_axis_name="s")

    @functools.partial(
        pl.kernel,
        mesh=mesh,
        out_type=jax.ShapeDtypeStruct((n_neurons, _B), jnp.float32),
        compiler_params=pltpu.CompilerParams(needs_layout_passes=False),
        scratch_types=(
            [pltpu.VMEM((_CH, _HASH), jnp.float32) for _ in range(_RING)]
            + [pltpu.VMEM((_CH, _B), jnp.int32) for _ in range(_RING)]
            + [pltpu.VMEM((_CH, _B), jnp.float32) for _ in range(_RING)]
            + [
                pltpu.SemaphoreType.DMA,
                pltpu.SemaphoreType.DMA,
                pltpu.SemaphoreType.DMA,
            ]
        ),
    )
    def gather_kernel(table_hbm, addr_hbm, out_hbm, *scratch):
        rows_v = scratch[:_RING]
        idx_v = scratch[_RING:2 * _RING]
        out_v = scratch[2 * _RING:3 * _RING]
        rsem, isem, osem = scratch[3 * _RING:]
        c = lax.axis_index("c")
        s = lax.axis_index("s")
        base = (s * 2 + c) * per_w

        for p in range(_RING - 1):
            pltpu.async_copy(table_hbm.at[pl.ds(base + p * _CH, _CH)],
                             rows_v[p], rsem)
            pltpu.async_copy(addr_hbm.at[pl.ds(base + p * _CH, _CH)],
                             idx_v[p], isem)

        def group(g, carry):
            for sl in range(_RING):
                ci = g * _RING + sl
                cf = ci + _RING - 1

                @pl.when(cf < n_chunks)
                def _():
                    pltpu.async_copy(
                        table_hbm.at[pl.ds(base + cf * _CH, _CH)],
                        rows_v[(sl - 1) % _RING], rsem)
                    pltpu.async_copy(
                        addr_hbm.at[pl.ds(base + cf * _CH, _CH)],
                        idx_v[(sl - 1) % _RING], isem)

                pltpu.make_async_copy(
                    table_hbm.at[pl.ds(base, _CH)], rows_v[sl], rsem).wait()
                pltpu.make_async_copy(
                    addr_hbm.at[pl.ds(base, _CH)], idx_v[sl], isem).wait()

                @pl.when(ci >= _RING)
                def _():
                    pltpu.make_async_copy(
                        out_v[sl], out_hbm.at[pl.ds(base, _CH)], osem).wait()

                rows = rows_v[sl]
                idx = idx_v[sl]
                ov = out_v[sl]

                for r in range(_CH):
                    rvec = jnp.full((16,), r, jnp.int32)

                    def gather16(v, _, *, r=r, rvec=rvec):
                        for u in range(8):
                            off = v * 128 + u * 16
                            cols = idx[r, pl.ds(off, 16)]
                            vals = plsc.load_gather(rows, [rvec, cols])
                            ov[r, pl.ds(off, 16)] = vals
                        return _

                    lax.fori_loop(0, _B // 128, gather16, 0, unroll=True)

                pltpu.async_copy(
                    ov, out_hbm.at[pl.ds(base + ci * _CH, _CH)], osem)
            return carry

        lax.fori_loop(0, n_chunks // _RING, group, 0)

        for sl in range(_RING):
            pltpu.make_async_copy(
                out_v[sl], out_hbm.at[pl.ds(base, _CH)], osem).wait()

    return gather_kernel


def kernel(window_bits, conn_state, conn_out, state_memory, output_memory):
    state_layer_input = _concat_call(window_bits)

    w2lo, w2hi = _w2_call(conn_out)

    addr1t = _addr1_call(window_bits, conn_state)
    out1t = _make_sc_gather(_N_ST)(state_memory, addr1t)

    addr2t = _addr2_call(out1t, w2lo, w2hi)
    out2t = _make_sc_gather(_N_OUT)(output_memory, addr2t)

    state_layer_output = _make_transpose(_N_ST)(out1t)
    output_layer_output = _make_transpose(_N_OUT)(out2t)

    return (state_layer_input, state_layer_output, state_layer_output,
            output_layer_output)
